# baseline (speedup 1.0000x reference)
import jax
import jax.numpy as jnp
from jax import lax
from jax.experimental import pallas as pl
from jax.experimental.pallas import tpu as pltpu
from jax.experimental.pallas import tpu_sc as plsc

N_USER = 10000
N_ITEM = 40000
N = N_USER + N_ITEM
NP = 50048
NH = NP // 2
D = 64
E = 800000
B = 1024

NC = 2
NS = 16
L = 16

KB = 128
SB = 8
SKB = SB * KB
EPAD = 819200
NSK = EPAD // SKB
EPAD2 = EPAD + NS * SKB
NZ_T = (NH + 16) // NS
NR_T = NH // NS

_mesh = plsc.VectorSubcoreMesh(
    core_axis_name="c", subcore_axis_name="s", num_cores=NC, num_subcores=NS)


def _spmm_body(ego_hbm, col_hbm, row_hbm, val_hbm, zz_hbm, prm_hbm, out_hbm,
               colv, rowv, valv, prmv, rbf0, rbf1, sc0, sc1, acc,
               gsem0, gsem1, ssem0, ssem1):
    c = lax.axis_index("c")
    s = lax.axis_index("s")
    rbf = (rbf0, rbf1)
    scf = (sc0, sc1)
    gsem = (gsem0, gsem1)
    ssem = (ssem0, ssem1)

    pltpu.sync_copy(prm_hbm, prmv)
    pv = prmv[pl.ds(0, L)]
    n_sk = jnp.where(c == 0, pv[0], pv[2])
    base_sk = jnp.where(c == 0, 0, pv[1])
    nsb_t = lax.div(n_sk + NS - 1, NS)

    pltpu.sync_copy(zz_hbm, acc.at[pl.ds(s * NZ_T, NZ_T)])
    plsc.subcore_barrier()

    rowbase = jnp.full((L,), c * NH, jnp.int32)
    dummy = jnp.full((L,), NH, jnp.int32)

    def scale(bbuf, fbuf, vbase):
        def grp(g, carry):
            vv = valv[pl.ds(vbase + g * L, L)]
            for u in range(L):
                kk = g * L + u
                bv = lax.broadcast(vv[u], (L,))
                for h in range(2):
                    ra, rb = plsc.unpack(bbuf[kk, pl.ds(2 * L * h, 2 * L)],
                                         format=plsc.PackFormat.INTERLEAVED,
                                         preferred_element_type=jnp.float32)
                    fbuf[kk, pl.ds(2 * L * h, L)] = ra * bv
                    fbuf[kk, pl.ds(2 * L * h + L, L)] = rb * bv
            return carry
        lax.fori_loop(0, KB // L, grp, 0)

    def superchunk(i, carry):
        ebase = (base_sk + s * nsb_t + i) * SKB
        pltpu.sync_copy(col_hbm.at[pl.ds(ebase, SKB)], colv)
        pltpu.sync_copy(row_hbm.at[pl.ds(ebase, SKB)], rowv)
        pltpu.sync_copy(val_hbm.at[pl.ds(ebase, SKB)], valv)

        def clamp(g, carry):
            rv = rowv[pl.ds(g * L, L)] - rowbase
            ok = (rv >= 0) & (rv < NH)
            rowv[pl.ds(g * L, L)] = jnp.where(ok, rv, dummy)
            return carry
        lax.fori_loop(0, SKB // L, clamp, 0)

        pend_g = [None, None]
        pend_s = [None, None]

        def fire_gather(b):
            p = b & 1
            d = pltpu.make_async_copy(
                ego_hbm.at[colv.at[pl.ds(b * KB, KB)]], rbf[p], gsem[p])
            d.start()
            pend_g[p] = d

        def drain_scale_scatter(b):
            p = b & 1
            pend_g[p].wait()
            scale(rbf[p], scf[p], b * KB)
            pend_s[p] = pltpu.async_copy(
                scf[p], acc.at[rowv.at[pl.ds(b * KB, KB)]], ssem[p], add=True)

        for b in range(SB):
            p = b & 1
            if b >= 2 and pend_s[p] is not None:
                pend_s[p].wait()
                pend_s[p] = None
            fire_gather(b)
            if b >= 1:
                drain_scale_scatter(b - 1)
        drain_scale_scatter(SB - 1)
        for p in range(2):
            if pend_s[p] is not None:
                pend_s[p].wait()
        return carry

    lax.fori_loop(0, nsb_t, superchunk, 0)
    plsc.subcore_barrier()

    pltpu.sync_copy(acc.at[pl.ds(s * NR_T, NR_T)],
                    out_hbm.at[pl.ds(c * NH + s * NR_T, NR_T)])
    plsc.subcore_barrier()


@jax.jit
def _spmm(egob, colp, rowp, valp, zz, prm):
    return pl.kernel(
        _spmm_body,
        out_type=jax.ShapeDtypeStruct((NP, D), jnp.float32),
        mesh=_mesh,
        scratch_types=[
            pltpu.VMEM((SKB,), jnp.int32),
            pltpu.VMEM((SKB,), jnp.int32),
            pltpu.VMEM((SKB,), jnp.float32),
            pltpu.VMEM((L,), jnp.int32),
            pltpu.VMEM((KB, D), jnp.bfloat16),
            pltpu.VMEM((KB, D), jnp.bfloat16),
            pltpu.VMEM((KB, D), jnp.float32),
            pltpu.VMEM((KB, D), jnp.float32),
            pltpu.VMEM_SHARED((NH + 16, D), jnp.float32),
            pltpu.SemaphoreType.DMA,
            pltpu.SemaphoreType.DMA,
            pltpu.SemaphoreType.DMA,
            pltpu.SemaphoreType.DMA,
        ],
        compiler_params=pltpu.CompilerParams(
            use_tc_tiling_on_sc=False, needs_layout_passes=False),
    )(egob, colp, rowp, valp, zz, prm)


BN = 6256


def _dense_body(side_ref, w_ref, b_ref, tot_ref, ego_ref, totout_ref):
    y = side_ref[...] @ w_ref[...] + b_ref[...]
    y = jnp.where(y >= 0, y, 0.2 * y)
    n2 = jnp.sum(y * y, axis=1, keepdims=True)
    nrm = y / jnp.maximum(jnp.sqrt(n2), 1e-12)
    totout_ref[...] = tot_ref[...] + nrm
    ego_ref[...] = y.astype(jnp.bfloat16)


@jax.jit
def _dense(side, w, b, total):
    return pl.pallas_call(
        _dense_body,
        grid=(NP // BN,),
        in_specs=[
            pl.BlockSpec((BN, D), lambda i: (i, 0)),
            pl.BlockSpec((D, D), lambda i: (0, 0)),
            pl.BlockSpec((1, D), lambda i: (0, 0)),
            pl.BlockSpec((BN, D), lambda i: (i, 0)),
        ],
        out_specs=[
            pl.BlockSpec((BN, D), lambda i: (i, 0)),
            pl.BlockSpec((BN, D), lambda i: (i, 0)),
        ],
        out_shape=[
            jax.ShapeDtypeStruct((NP, D), jnp.bfloat16),
            jax.ShapeDtypeStruct((NP, D), jnp.float32),
        ],
    )(side, w, b, total)


BPW = B // (NC * NS)


def _gather_body(tot_hbm, users_hbm, out_hbm, idxv, rowsv, sem):
    wid = lax.axis_index("s") * NC + lax.axis_index("c")
    base = wid * BPW
    pltpu.sync_copy(users_hbm.at[pl.ds(base, BPW)], idxv)
    pltpu.async_copy(tot_hbm.at[idxv], rowsv, sem).wait()
    pltpu.sync_copy(rowsv, out_hbm.at[pl.ds(base, BPW)])


@jax.jit
def _gather(total, users):
    return pl.kernel(
        _gather_body,
        out_type=jax.ShapeDtypeStruct((B, D), jnp.float32),
        mesh=_mesh,
        scratch_types=[
            pltpu.VMEM((BPW,), jnp.int32),
            pltpu.VMEM((BPW, D), jnp.float32),
            pltpu.SemaphoreType.DMA,
        ],
        compiler_params=pltpu.CompilerParams(use_tc_tiling_on_sc=False),
    )(total, users)


def kernel(users, user_emb, item_emb, adj_row, adj_col, adj_val,
           W_gc_0, b_gc_0, W_gc_1, b_gc_1, W_gc_2, b_gc_2):
    users = users.astype(jnp.int32)
    col = adj_col.astype(jnp.int32)
    row = adj_row.astype(jnp.int32)
    val = adj_val.astype(jnp.float32)

    pad = EPAD2 - E
    colp = jnp.concatenate([col, jnp.zeros((pad,), jnp.int32)])
    rowp = jnp.concatenate([row, jnp.full((pad,), N - 1, jnp.int32)])
    valp = jnp.concatenate([val, jnp.zeros((pad,), jnp.float32)])
    zz = jnp.zeros((NZ_T, D), jnp.float32)

    m = jnp.searchsorted(row, NH).astype(jnp.int32)
    hi0 = lax.div(m + SKB - 1, jnp.int32(SKB))
    lo1 = lax.div(m, jnp.int32(SKB))
    prm = jnp.zeros((L,), jnp.int32).at[0].set(hi0).at[1].set(lo1)
    prm = prm.at[2].set(NSK - lo1)

    quarter = jnp.concatenate([jnp.arange(0, 32, 2), jnp.arange(1, 32, 2)])
    colperm = jnp.concatenate([quarter, quarter + 32])

    ego = jnp.concatenate(
        [user_emb, item_emb, jnp.zeros((NP - N, D), jnp.float32)], axis=0)
    total = ego
    egob = ego.astype(jnp.bfloat16)

    for w, bb in ((W_gc_0, b_gc_0), (W_gc_1, b_gc_1), (W_gc_2, b_gc_2)):
        side = _spmm(egob, colp, rowp, valp, zz, prm)
        egob, total = _dense(side, w[colperm, :], bb, total)

    return _gather(total, users)
